# baseline (speedup 1.0000x reference)
import functools

import jax
import jax.numpy as jnp
from jax import lax
from jax.experimental import pallas as pl
from jax.experimental.pallas import tpu as pltpu


def _gap_matmul_kernel(x_ref, w_ref, o_ref):
    o_ref[...] = lax.dot_general(
        x_ref[...], w_ref[...],
        dimension_numbers=(((1,), (0,)), ((), ())),
        preferred_element_type=jnp.float32,
    ).astype(o_ref.dtype)


def _gap_rowsum_kernel(x_ref, o_ref, *, inv_hw):
    o_ref[...] = (
        jnp.sum(x_ref[...].astype(jnp.float32), axis=-1, keepdims=True) * inv_hw
    ).astype(o_ref.dtype)


def kernel(x):
    n, c, h, w = x.shape
    hw = h * w
    rows = n * c

    g = 1
    for cand in (16, 8, 4, 2):
        if rows % cand == 0 and cand * hw <= 1024:
            g = cand
            break

    if g == 1 or hw > 1024:
        x2 = x.reshape(rows, hw)
        tr = 4096
        grid = (pl.cdiv(rows, tr),)
        out = pl.pallas_call(
            functools.partial(_gap_rowsum_kernel, inv_hw=1.0 / hw),
            out_shape=jax.ShapeDtypeStruct((rows, 1), x.dtype),
            grid=grid,
            in_specs=[pl.BlockSpec((tr, hw), lambda i: (i, 0))],
            out_specs=pl.BlockSpec((tr, 1), lambda i: (i, 0)),
            compiler_params=pltpu.CompilerParams(
                dimension_semantics=("parallel",),
                vmem_limit_bytes=32 << 20,
            ),
        )(x2)
        return out.reshape(n, c)

    rows2 = rows // g
    b = g * hw
    x2 = x.reshape(rows2, b)

    e = jnp.arange(b, dtype=jnp.int32)[:, None]
    j = jnp.arange(g, dtype=jnp.int32)[None, :]
    wmat = jnp.where(e // hw == j, jnp.float32(1.0 / hw), jnp.float32(0.0))

    tr = 2048
    while rows2 % tr != 0:
        tr //= 2
    grid = (rows2 // tr,)

    out = pl.pallas_call(
        _gap_matmul_kernel,
        out_shape=jax.ShapeDtypeStruct((rows2, g), x.dtype),
        grid=grid,
        in_specs=[
            pl.BlockSpec((tr, b), lambda i: (i, 0)),
            pl.BlockSpec((b, g), lambda i: (0, 0)),
        ],
        out_specs=pl.BlockSpec((tr, g), lambda i: (i, 0)),
        compiler_params=pltpu.CompilerParams(
            dimension_semantics=("parallel",),
            vmem_limit_bytes=32 << 20,
        ),
        cost_estimate=pl.CostEstimate(
            flops=2 * rows2 * b * g,
            transcendentals=0,
            bytes_accessed=rows2 * b * 4 + rows2 * g * 4,
        ),
    )(x2, wmat)
    return out.reshape(n, c)


# speedup vs baseline: 47.4855x; 47.4855x over previous
import functools

import jax
import jax.numpy as jnp
from jax.experimental import pallas as pl
from jax.experimental.pallas import tpu as pltpu


def _plane_sum_kernel(x_ref, o_ref, *, inv_hw):
    o_ref[...] = (
        jnp.sum(x_ref[...].astype(jnp.float32), axis=0) * inv_hw
    ).astype(o_ref.dtype)


def kernel(x):
    n, c, h, w = x.shape
    hw = h * w

    xt = jnp.transpose(x, (2, 3, 0, 1)).reshape(hw, n, c)

    tn = n
    for cand in (16, 8, 4, 2, 1):
        if n % cand == 0 and hw * cand * c * 4 <= (12 << 20):
            tn = cand
            break
    grid = (n // tn,)

    return pl.pallas_call(
        functools.partial(_plane_sum_kernel, inv_hw=1.0 / hw),
        out_shape=jax.ShapeDtypeStruct((n, c), x.dtype),
        grid=grid,
        in_specs=[pl.BlockSpec((hw, tn, c), lambda i: (0, i, 0))],
        out_specs=pl.BlockSpec((tn, c), lambda i: (i, 0)),
        compiler_params=pltpu.CompilerParams(
            dimension_semantics=("parallel",),
            vmem_limit_bytes=32 << 20,
        ),
        cost_estimate=pl.CostEstimate(
            flops=n * c * hw,
            transcendentals=0,
            bytes_accessed=n * c * hw * 4 + n * c * 4,
        ),
    )(xt)
